# Initial kernel scaffold; baseline (speedup 1.0000x reference)
import jax
import jax.numpy as jnp
from jax.experimental import pallas as pl


def kernel(query, support, W1, b1, g1, t1, W2, b2, g2, t2, W3, b3, g3, t3, W4, b4, g4, t4):
    raise NotImplementedError("write your pallas kernel here")


# speedup vs baseline: 2.0217x; 2.0217x over previous
import jax
import jax.numpy as jnp
from jax.experimental import pallas as pl
from jax.experimental.pallas import tpu as pltpu

F32 = jnp.float32

_B, _NQ, _WAY, _SHOT = 2, 15, 5, 5
_H0 = 84
_HW1, _W1 = 84 * 84, 84
_HW2, _W2 = 42 * 42, 42
_HW3, _W3 = 21 * 21, 21
_D = 64
_NIMG = _B * _NQ + _B * _WAY * _SHOT
_M = _SHOT * _HW3


def _conv9(src_ref, taps, P, W, bias):
    pad = W + 1
    row = jax.lax.broadcasted_iota(jnp.int32, (P, 1), 0)
    j = jax.lax.rem(row, W)
    acc = jnp.broadcast_to(bias[None, :], (P, 64)).astype(F32)
    t = 0
    for di in (-1, 0, 1):
        for dj in (-1, 0, 1):
            off = di * W + dj
            xs = src_ref[pad + off: pad + off + P, :]
            if dj == -1:
                xs = jnp.where(j >= 1, xs, 0.0)
            elif dj == 1:
                xs = jnp.where(j <= W - 2, xs, 0.0)
            acc = acc + jnp.dot(xs, taps[t], preferred_element_type=F32)
            t += 1
    return acc


def _leaky(x):
    return jnp.where(x >= 0, x, 0.2 * x)


def _pool_into(y, tmp_ref, dst_ref, W, Wn):
    tmp_ref[...] = y
    a = jnp.maximum(tmp_ref[0::2, :], tmp_ref[1::2, :])
    padn = Wn + 1
    for i2 in range(W // 2):
        r0 = (2 * i2) * Wn
        r1 = (2 * i2 + 1) * Wn
        blk = jnp.maximum(a[r0: r0 + Wn, :], a[r1: r1 + Wn, :])
        dst_ref[padn + i2 * Wn: padn + (i2 + 1) * Wn, :] = blk


def _zero_pads(ref, pad, P):
    ref[0:pad, :] = jnp.zeros((pad, ref.shape[1]), F32)
    ref[pad + P:, :] = jnp.zeros((ref.shape[0] - pad - P, ref.shape[1]), F32)


def _enc_body(x_ref, w1_ref, w_ref, b_ref, o_ref, s1, s2, s3, s4, t1, t2):
    p1 = _W1 + 1
    _zero_pads(s1, p1, _HW1)
    s1[p1: p1 + _HW1, :] = x_ref[0]

    taps1 = [w1_ref[t] for t in range(9)]
    y = _leaky(_conv9(s1, taps1, _HW1, _W1, b_ref[0]))
    _zero_pads(s2, _W2 + 1, _HW2)
    _pool_into(y, t1, s2, _W1, _W2)

    taps2 = [w_ref[0, t] for t in range(9)]
    y = _leaky(_conv9(s2, taps2, _HW2, _W2, b_ref[1]))
    _zero_pads(s3, _W3 + 1, _HW3)
    _pool_into(y, t2, s3, _W2, _W3)

    taps3 = [w_ref[1, t] for t in range(9)]
    y = _leaky(_conv9(s3, taps3, _HW3, _W3, b_ref[2]))
    _zero_pads(s4, _W3 + 1, _HW3)
    s4[_W3 + 1: _W3 + 1 + _HW3, :] = y

    taps4 = [w_ref[2, t] for t in range(9)]
    y = _leaky(_conv9(s4, taps4, _HW3, _W3, b_ref[3]))

    n = jnp.sqrt(jnp.sum(y * y, axis=1, keepdims=True))
    o_ref[0] = y / jnp.clip(n, 1e-12)


def _score_body(q_ref, s_ref, o_ref):
    q = q_ref[0]
    s = s_ref[0]
    sim = jnp.dot(q, s, preferred_element_type=F32)
    neg = jnp.float32(-jnp.inf)
    m1 = jnp.max(sim, axis=1, keepdims=True)
    e1 = sim == m1
    c1 = jnp.sum(e1.astype(F32), axis=1, keepdims=True)
    sim2 = jnp.where(e1, neg, sim)
    m2 = jnp.max(sim2, axis=1, keepdims=True)
    e2 = sim2 == m2
    c2 = jnp.sum(e2.astype(F32), axis=1, keepdims=True)
    sim3 = jnp.where(e2, neg, sim2)
    m3 = jnp.max(sim3, axis=1, keepdims=True)
    second = jnp.where(c1 >= 2, m1, m2)
    third = jnp.where(c1 >= 3, m1, jnp.where(c1 + c2 >= 3, m2, m3))
    o_ref[0] = jnp.sum(m1 + second + third, axis=0, keepdims=True)


def kernel(query, support, W1, b1, g1, t1, W2, b2, g2, t2, W3, b3, g3, t3, W4, b4, g4, t4):
    def prep(W, b, g, t):
        Wf = W * g[:, None, None, None]
        taps = jnp.transpose(Wf, (2, 3, 1, 0))
        taps = taps.reshape(9, W.shape[1], 64)
        return taps, b * g + t

    w1p, bias1 = prep(W1, b1, g1, t1)
    w2p, bias2 = prep(W2, b2, g2, t2)
    w3p, bias3 = prep(W3, b3, g3, t3)
    w4p, bias4 = prep(W4, b4, g4, t4)
    wp = jnp.stack([w2p, w3p, w4p])
    biases = jnp.stack([bias1, bias2, bias3, bias4])

    q_imgs = jnp.transpose(query.reshape(-1, 3, _H0, _H0), (0, 2, 3, 1))
    s_imgs = jnp.transpose(support.reshape(-1, 3, _H0, _H0), (0, 2, 3, 1))
    imgs = jnp.concatenate([q_imgs, s_imgs], 0).reshape(_NIMG, _HW1, 3)

    feats = pl.pallas_call(
        _enc_body,
        grid=(_NIMG,),
        in_specs=[
            pl.BlockSpec((1, _HW1, 3), lambda i: (i, 0, 0)),
            pl.BlockSpec((9, 3, 64), lambda i: (0, 0, 0)),
            pl.BlockSpec((3, 9, 64, 64), lambda i: (0, 0, 0, 0)),
            pl.BlockSpec((4, 64), lambda i: (0, 0)),
        ],
        out_specs=pl.BlockSpec((1, _HW3, _D), lambda i: (i, 0, 0)),
        out_shape=jax.ShapeDtypeStruct((_NIMG, _HW3, _D), F32),
        scratch_shapes=[
            pltpu.VMEM((_HW1 + 2 * (_W1 + 1), 3), F32),
            pltpu.VMEM((_HW2 + 2 * (_W2 + 1), 64), F32),
            pltpu.VMEM((_HW3 + 2 * (_W3 + 1), 64), F32),
            pltpu.VMEM((_HW3 + 2 * (_W3 + 1), 64), F32),
            pltpu.VMEM((_HW1, 64), F32),
            pltpu.VMEM((_HW2, 64), F32),
        ],
        compiler_params=pltpu.CompilerParams(
            dimension_semantics=("arbitrary",)),
    )(imgs, w1p, wp, biases)

    qn = feats[: _B * _NQ]
    sn = feats[_B * _NQ:].reshape(_B, _WAY, _SHOT * _HW3, _D)
    st = jnp.transpose(sn, (0, 1, 3, 2)).reshape(_B * _WAY, _D, _M)

    nq_total = _B * _NQ
    scores = pl.pallas_call(
        _score_body,
        grid=(nq_total, _WAY),
        in_specs=[
            pl.BlockSpec((1, _HW3, _D), lambda qi, c: (qi, 0, 0)),
            pl.BlockSpec((1, _D, _M), lambda qi, c: ((qi // _NQ) * _WAY + c, 0, 0)),
        ],
        out_specs=pl.BlockSpec((1, 1, 1), lambda qi, c: (qi * _WAY + c, 0, 0)),
        out_shape=jax.ShapeDtypeStruct((nq_total * _WAY, 1, 1), F32),
        compiler_params=pltpu.CompilerParams(
            dimension_semantics=("arbitrary", "arbitrary")),
    )(qn, st)

    return scores.reshape(nq_total, _WAY)


# speedup vs baseline: 3.2796x; 1.6222x over previous
import jax
import jax.numpy as jnp
from jax.experimental import pallas as pl
from jax.experimental.pallas import tpu as pltpu

F32 = jnp.float32

_B, _NQ, _WAY, _SHOT = 2, 15, 5, 5
_H1, _W1P = 84, 88
_P1 = _H1 * _W1P
_H2, _W2P = 42, 46
_P2 = _H2 * _W2P
_H3, _W3P = 21, 25
_P3 = _H3 * _W3P
_HW = 21 * 21
_D = 64
_NIMG = _B * _NQ + _B * _WAY * _SHOT
_M = _SHOT * _HW

_LPAD = 128
_RPAD2, _RPAD3 = 48, 32

_OFF1 = [di * _W1P + dj for di in (-1, 0, 1) for dj in (-1, 0, 1)]
_OFF2 = [di * _W2P + dj for di in (-1, 0, 1) for dj in (-1, 0, 1)]
_OFF3 = [di * _W3P + dj for di in (-1, 0, 1) for dj in (-1, 0, 1)]


def _leaky(x):
    return jnp.where(x >= 0, x, 0.2 * x)


def _conv9(src_ref, w_ref, li, offs, rpad, P, bias):
    acc = jnp.broadcast_to(bias[None, :], (P, 64)).astype(F32)
    for t, off in enumerate(offs):
        xs = src_ref[rpad + off: rpad + off + P, :]
        acc = acc + jnp.dot(xs, w_ref[li, t], preferred_element_type=F32)
    return acc


def _enc_body(x_ref, w1_ref, w_ref, b_ref, o_ref, s1, s2, s3, s4, t1, t2):
    s1[...] = jnp.zeros(s1.shape, F32)
    s2[...] = jnp.zeros(s2.shape, F32)
    s3[...] = jnp.zeros(s3.shape, F32)
    s4[...] = jnp.zeros(s4.shape, F32)

    s1[0:3, _LPAD: _LPAD + _P1] = x_ref[0]
    xk = jnp.concatenate(
        [s1[:, _LPAD + off: _LPAD + off + _P1] for off in _OFF1], axis=0)
    y = jax.lax.dot_general(xk, w1_ref[...], (((0,), (0,)), ((), ())),
                            preferred_element_type=F32)
    y = _leaky(y + b_ref[0][None, :])

    t1[...] = y
    a = jnp.maximum(t1[0::2, :], t1[1::2, :])
    half1 = _W1P // 2
    for i2 in range(_H2):
        r0 = (2 * i2) * half1
        blk = jnp.maximum(a[r0: r0 + half1, :], a[r0 + half1: r0 + 2 * half1, :])
        base = _RPAD2 + i2 * _W2P
        s2[base + 2: base + 2 + _H2, :] = blk[1: 1 + _H2, :]

    y = _leaky(_conv9(s2, w_ref, 0, _OFF2, _RPAD2, _P2, b_ref[1]))
    t2[...] = y
    a = jnp.maximum(t2[0::2, :], t2[1::2, :])
    half2 = _W2P // 2
    for i2 in range(_H3):
        r0 = (2 * i2) * half2
        blk = jnp.maximum(a[r0: r0 + half2, :], a[r0 + half2: r0 + 2 * half2, :])
        base = _RPAD3 + i2 * _W3P
        s3[base + 2: base + 2 + _H3, :] = blk[1: 1 + _H3, :]

    y = _leaky(_conv9(s3, w_ref, 1, _OFF3, _RPAD3, _P3, b_ref[2]))
    for i in range(_H3):
        s4[_RPAD3 + i * _W3P + 2: _RPAD3 + i * _W3P + 2 + _H3, :] = \
            y[i * _W3P + 2: i * _W3P + 2 + _H3, :]

    y = _leaky(_conv9(s4, w_ref, 2, _OFF3, _RPAD3, _P3, b_ref[3]))
    n = jnp.sqrt(jnp.sum(y * y, axis=1, keepdims=True))
    yn = y / jnp.clip(n, 1e-12)
    for i in range(_H3):
        o_ref[0, i * _H3: (i + 1) * _H3, :] = \
            yn[i * _W3P + 2: i * _W3P + 2 + _H3, :]


def _score_body(q_ref, s_ref, o_ref):
    q = q_ref[0]
    s = s_ref[0]
    sim = jnp.dot(q, s, preferred_element_type=F32)
    neg = jnp.float32(-jnp.inf)
    m1 = jnp.max(sim, axis=1, keepdims=True)
    e1 = sim == m1
    c1 = jnp.sum(e1.astype(F32), axis=1, keepdims=True)
    sim2 = jnp.where(e1, neg, sim)
    m2 = jnp.max(sim2, axis=1, keepdims=True)
    e2 = sim2 == m2
    c2 = jnp.sum(e2.astype(F32), axis=1, keepdims=True)
    sim3 = jnp.where(e2, neg, sim2)
    m3 = jnp.max(sim3, axis=1, keepdims=True)
    second = jnp.where(c1 >= 2, m1, m2)
    third = jnp.where(c1 >= 3, m1, jnp.where(c1 + c2 >= 3, m2, m3))
    o_ref[0] = jnp.sum(m1 + second + third, axis=0, keepdims=True)


def kernel(query, support, W1, b1, g1, t1, W2, b2, g2, t2, W3, b3, g3, t3, W4, b4, g4, t4):
    def prep(W, b, g, t):
        Wf = W * g[:, None, None, None]
        taps = jnp.transpose(Wf, (2, 3, 1, 0))
        taps = taps.reshape(9, W.shape[1], 64)
        return taps, b * g + t

    w1p, bias1 = prep(W1, b1, g1, t1)
    w1k = jnp.pad(w1p, ((0, 0), (0, 5), (0, 0))).reshape(72, 64)
    w2p, bias2 = prep(W2, b2, g2, t2)
    w3p, bias3 = prep(W3, b3, g3, t3)
    w4p, bias4 = prep(W4, b4, g4, t4)
    wp = jnp.stack([w2p, w3p, w4p])
    biases = jnp.stack([bias1, bias2, bias3, bias4])

    q_imgs = query.reshape(-1, 3, _H1, _H1)
    s_imgs = support.reshape(-1, 3, _H1, _H1)
    imgs = jnp.concatenate([q_imgs, s_imgs], 0)
    imgs = jnp.pad(imgs, ((0, 0), (0, 0), (0, 0), (2, 2))).reshape(_NIMG, 3, _P1)

    feats = pl.pallas_call(
        _enc_body,
        grid=(_NIMG,),
        in_specs=[
            pl.BlockSpec((1, 3, _P1), lambda i: (i, 0, 0)),
            pl.BlockSpec((72, 64), lambda i: (0, 0)),
            pl.BlockSpec((3, 9, 64, 64), lambda i: (0, 0, 0, 0)),
            pl.BlockSpec((4, 64), lambda i: (0, 0)),
        ],
        out_specs=pl.BlockSpec((1, _HW, _D), lambda i: (i, 0, 0)),
        out_shape=jax.ShapeDtypeStruct((_NIMG, _HW, _D), F32),
        scratch_shapes=[
            pltpu.VMEM((8, _P1 + 2 * _LPAD), F32),
            pltpu.VMEM((_P2 + 2 * _RPAD2, 64), F32),
            pltpu.VMEM((_P3 + 2 * _RPAD3, 64), F32),
            pltpu.VMEM((_P3 + 2 * _RPAD3, 64), F32),
            pltpu.VMEM((_P1, 64), F32),
            pltpu.VMEM((_P2, 64), F32),
        ],
        compiler_params=pltpu.CompilerParams(
            dimension_semantics=("arbitrary",)),
    )(imgs, w1k, wp, biases)

    qn = feats[: _B * _NQ]
    sn = feats[_B * _NQ:].reshape(_B, _WAY, _M, _D)
    st = jnp.transpose(sn, (0, 1, 3, 2)).reshape(_B * _WAY, _D, _M)

    nq_total = _B * _NQ
    scores = pl.pallas_call(
        _score_body,
        grid=(nq_total, _WAY),
        in_specs=[
            pl.BlockSpec((1, _HW, _D), lambda qi, c: (qi, 0, 0)),
            pl.BlockSpec((1, _D, _M), lambda qi, c: ((qi // _NQ) * _WAY + c, 0, 0)),
        ],
        out_specs=pl.BlockSpec((1, 1, 1), lambda qi, c: (qi * _WAY + c, 0, 0)),
        out_shape=jax.ShapeDtypeStruct((nq_total * _WAY, 1, 1), F32),
        compiler_params=pltpu.CompilerParams(
            dimension_semantics=("arbitrary", "arbitrary")),
    )(qn, st)

    return scores.reshape(nq_total, _WAY)
